# baseline (speedup 1.0000x reference)
import jax
import jax.numpy as jnp
from jax.experimental import pallas as pl
from jax.experimental.pallas import tpu as pltpu

_NBISECT = 6
_NREFINE = 2


def _rowsum(v, ones):
    return jax.lax.dot_general(
        v, ones, (((1,), (0,)), ((), ())),
        preferred_element_type=jnp.float32)


def _loss_block(x_ref, t_ref, out_ref):
    x = x_ref[...]
    tgt = t_ref[...]
    ones = jnp.ones((x.shape[1], 1), jnp.float32)
    m = jnp.max(x, axis=1, keepdims=True)
    xs = (x - m) * 0.5

    lo = jnp.full_like(m, -1.0)
    hi = jnp.zeros_like(m)
    for _ in range(_NBISECT):
        mid = (lo + hi) * 0.5
        r = jnp.maximum(xs - mid, 0.0)
        f = _rowsum(r * r, ones)
        gt = f > 1.0
        lo = jnp.where(gt, mid, lo)
        hi = jnp.where(gt, hi, mid)
    tau = (lo + hi) * 0.5

    for _ in range(_NREFINE):
        mk = jnp.where(xs > tau, 1.0, 0.0)
        mxs = mk * xs
        k = _rowsum(mk, ones)
        s1 = _rowsum(mxs, ones)
        s2 = _rowsum(mxs * xs, ones)
        mean = s1 / k
        delta = (1.0 - (s2 - s1 * mean)) / k
        tau = mean - jnp.sqrt(jnp.maximum(delta, 0.0))

    r = jnp.maximum(xs - tau, 0.0)
    p = r * r
    s3 = _rowsum(p * r, ones)
    iota = jax.lax.broadcasted_iota(jnp.int32, x.shape, 1)
    onehot = jnp.where(iota == tgt, 1.0, 0.0)
    spx = _rowsum((p - onehot) * x, ones)
    loss = (1.0 - s3) * (1.0 / 0.75) + spx
    out_ref[...] = jnp.reshape(jnp.sum(loss), (1, 1, 1))


def kernel(input, target):
    n, c = input.shape
    rows = 256 if n % 256 == 0 else n
    grid = n // rows
    tgt = target.astype(jnp.int32).reshape(n, 1)
    partials = pl.pallas_call(
        _loss_block,
        grid=(grid,),
        in_specs=[
            pl.BlockSpec((rows, c), lambda i: (i, 0)),
            pl.BlockSpec((rows, 1), lambda i: (i, 0)),
        ],
        out_specs=pl.BlockSpec((1, 1, 1), lambda i: (i, 0, 0)),
        out_shape=jax.ShapeDtypeStruct((grid, 1, 1), jnp.float32),
        compiler_params=pltpu.CompilerParams(
            dimension_semantics=("parallel",),
        ),
    )(input, tgt)
    return jnp.sum(partials) / float(n)


# speedup vs baseline: 1.6275x; 1.6275x over previous
import jax
import jax.numpy as jnp
from jax.experimental import pallas as pl
from jax.experimental.pallas import tpu as pltpu

_NBISECT = 6
_NREFINE = 2


def _rowsum(v, ones):
    del ones
    return jnp.sum(v, axis=1, keepdims=True)


def _loss_block(x_ref, t_ref, out_ref):
    x = x_ref[...]
    tgt = t_ref[...]
    ones = jnp.ones((x.shape[1], 1), jnp.float32)
    m = jnp.max(x, axis=1, keepdims=True)
    xs = (x - m) * 0.5

    lo = jnp.full_like(m, -1.0)
    hi = jnp.zeros_like(m)
    for _ in range(_NBISECT):
        mid = (lo + hi) * 0.5
        r = jnp.maximum(xs - mid, 0.0)
        f = _rowsum(r * r, ones)
        gt = f > 1.0
        lo = jnp.where(gt, mid, lo)
        hi = jnp.where(gt, hi, mid)
    tau = (lo + hi) * 0.5

    for _ in range(_NREFINE):
        mk = jnp.where(xs > tau, 1.0, 0.0)
        mxs = mk * xs
        k = _rowsum(mk, ones)
        s1 = _rowsum(mxs, ones)
        s2 = _rowsum(mxs * xs, ones)
        mean = s1 / k
        delta = (1.0 - (s2 - s1 * mean)) / k
        tau = mean - jnp.sqrt(jnp.maximum(delta, 0.0))

    r = jnp.maximum(xs - tau, 0.0)
    p = r * r
    s3 = _rowsum(p * r, ones)
    iota = jax.lax.broadcasted_iota(jnp.int32, x.shape, 1)
    onehot = jnp.where(iota == tgt, 1.0, 0.0)
    spx = _rowsum((p - onehot) * x, ones)
    loss = (1.0 - s3) * (1.0 / 0.75) + spx
    out_ref[...] = jnp.reshape(jnp.sum(loss), (1, 1, 1))


def kernel(input, target):
    n, c = input.shape
    rows = 256 if n % 256 == 0 else n
    grid = n // rows
    tgt = target.astype(jnp.int32).reshape(n, 1)
    partials = pl.pallas_call(
        _loss_block,
        grid=(grid,),
        in_specs=[
            pl.BlockSpec((rows, c), lambda i: (i, 0)),
            pl.BlockSpec((rows, 1), lambda i: (i, 0)),
        ],
        out_specs=pl.BlockSpec((1, 1, 1), lambda i: (i, 0, 0)),
        out_shape=jax.ShapeDtypeStruct((grid, 1, 1), jnp.float32),
        compiler_params=pltpu.CompilerParams(
            dimension_semantics=("parallel",),
        ),
    )(input, tgt)
    return jnp.sum(partials) / float(n)
